# baseline (speedup 1.0000x reference)
import functools

import jax
import jax.numpy as jnp
from jax import lax
from jax.experimental import pallas as pl
from jax.experimental.pallas import tpu as pltpu
from jax.experimental.pallas import tpu_sc as plsc

D = 1024
SEQ = 4096
BATCH = 4
NW = 32
SW = SEQ // NW
T = 16
NT = SW // T
LANES = 16
NI = NT * BATCH

_mesh = plsc.VectorSubcoreMesh(core_axis_name="c", subcore_axis_name="s")


@functools.partial(
    pl.kernel,
    mesh=_mesh,
    out_type=jax.ShapeDtypeStruct((BATCH * SEQ, D), jnp.float32),
    scratch_types=[
        pltpu.VMEM((2, T, D), jnp.float32),
        pltpu.VMEM((2, T, D), jnp.float32),
        pltpu.VMEM((2, T, D), jnp.float32),
        pltpu.VMEM((2, T), jnp.int32),
        pltpu.VMEM((LANES,), jnp.int32),
        pltpu.SemaphoreType.DMA((2,)),
        pltpu.SemaphoreType.DMA((2,)),
        pltpu.SemaphoreType.DMA((2,)),
    ],
)
def _sc_pe_add(x_hbm, pe_hbm, sp_hbm, out_hbm, pebuf, xbuf, obuf, idxbuf,
               spbuf, gsem, xsem, osem):
    wid = lax.axis_index("s") * 2 + lax.axis_index("c")
    s0 = wid * SW
    pltpu.sync_copy(sp_hbm, spbuf)
    vsp = spbuf[...]

    def pe_gather(t):
        slot = lax.rem(t, 2)
        idxbuf[slot, :] = vsp + lax.iota(jnp.int32, LANES) + (s0 + t * T)
        return pltpu.make_async_copy(
            pe_hbm.at[idxbuf.at[slot]], pebuf.at[slot], gsem.at[slot]
        )

    def pe_wait(t):
        slot = lax.rem(t, 2)
        return pltpu.make_async_copy(
            pe_hbm.at[idxbuf.at[slot]], pebuf.at[slot], gsem.at[slot]
        )

    def x_copy(k):
        t = lax.div(k, BATCH)
        b = lax.rem(k, BATCH)
        slot = lax.rem(k, 2)
        return pltpu.make_async_copy(
            x_hbm.at[pl.ds(b * SEQ + s0 + t * T, T)],
            xbuf.at[slot],
            xsem.at[slot],
        )

    def o_copy(k):
        t = lax.div(k, BATCH)
        b = lax.rem(k, BATCH)
        slot = lax.rem(k, 2)
        return pltpu.make_async_copy(
            obuf.at[slot],
            out_hbm.at[pl.ds(b * SEQ + s0 + t * T, T)],
            osem.at[slot],
        )

    pe_gather(jnp.int32(0)).start()
    x_copy(jnp.int32(0)).start()
    x_copy(jnp.int32(1)).start()

    def item(k, _):
        t = lax.div(k, BATCH)
        b = lax.rem(k, BATCH)
        slot = lax.rem(k, 2)
        pslot = lax.rem(t, 2)

        @pl.when(b == 0)
        def _():
            pe_wait(t).wait()

        @pl.when(jnp.logical_and(b == 3, t + 1 < NT))
        def _():
            pe_gather(t + 1).start()

        @pl.when(k >= 2)
        def _():
            o_copy(k - 2).wait()

        x_copy(k).wait()

        @plsc.parallel_loop(0, T, step=1, unroll=2)
        def add_row(r):
            for k2 in range(D // LANES):
                sl = pl.ds(k2 * LANES, LANES)
                obuf[slot, r, sl] = xbuf[slot, r, sl] + pebuf[pslot, r, sl]
        o_copy(k).start()

        @pl.when(k + 2 < NI)
        def _():
            x_copy(k + 2).start()

        return 0

    lax.fori_loop(0, NI, item, 0)
    o_copy(jnp.int32(NI - 2)).wait()
    o_copy(jnp.int32(NI - 1)).wait()


@jax.jit
def _pe_add(x, pos_embedding, sp16):
    batch, seq, d = x.shape
    xf = x.reshape(batch * seq, d)
    out = _sc_pe_add(xf, pos_embedding, sp16)
    return out.reshape(x.shape)


def kernel(x, pos_embedding, start_pos):
    sp16 = jnp.full((LANES,), start_pos, dtype=jnp.int32)
    return _pe_add(x, pos_embedding, sp16)


# speedup vs baseline: 1.6670x; 1.6670x over previous
import functools

import jax
import jax.numpy as jnp
from jax import lax
from jax.experimental import pallas as pl
from jax.experimental.pallas import tpu as pltpu
from jax.experimental.pallas import tpu_sc as plsc

D = 1024
SEQ = 4096
BATCH = 4
NW = 32
SW = SEQ // NW
T = 16
NT = SW // T
LANES = 16
NI = NT * BATCH

_mesh = plsc.VectorSubcoreMesh(core_axis_name="c", subcore_axis_name="s")


@functools.partial(
    pl.kernel,
    mesh=_mesh,
    out_type=jax.ShapeDtypeStruct((BATCH * SEQ, D), jnp.float32),
    scratch_types=[
        pltpu.VMEM((2, T, D), jnp.float32),
        pltpu.VMEM((3, T, D), jnp.float32),
        pltpu.VMEM((2, T, D), jnp.float32),
        pltpu.VMEM((2, T), jnp.int32),
        pltpu.VMEM((LANES,), jnp.int32),
        pltpu.SemaphoreType.DMA((2,)),
        pltpu.SemaphoreType.DMA((3,)),
        pltpu.SemaphoreType.DMA((2,)),
    ],
)
def _sc_pe_add(x_hbm, pe_hbm, sp_hbm, out_hbm, pebuf, xbuf, obuf, idxbuf,
               spbuf, gsem, xsem, osem):
    wid = lax.axis_index("s") * 2 + lax.axis_index("c")
    s0 = wid * SW
    pltpu.sync_copy(sp_hbm, spbuf)
    vsp = spbuf[...]

    def pe_gather(t):
        slot = lax.rem(t, 2)
        idxbuf[slot, :] = vsp + lax.iota(jnp.int32, LANES) + (s0 + t * T)
        return pltpu.make_async_copy(
            pe_hbm.at[idxbuf.at[slot]], pebuf.at[slot], gsem.at[slot]
        )

    def pe_wait(t):
        slot = lax.rem(t, 2)
        return pltpu.make_async_copy(
            pe_hbm.at[idxbuf.at[slot]], pebuf.at[slot], gsem.at[slot]
        )

    def x_copy(k):
        t = lax.div(k, BATCH)
        b = lax.rem(k, BATCH)
        slot = lax.rem(k, 3)
        return pltpu.make_async_copy(
            x_hbm.at[pl.ds(b * SEQ + s0 + t * T, T)],
            xbuf.at[slot],
            xsem.at[slot],
        )

    def o_copy(k):
        t = lax.div(k, BATCH)
        b = lax.rem(k, BATCH)
        slot = lax.rem(k, 2)
        return pltpu.make_async_copy(
            obuf.at[slot],
            out_hbm.at[pl.ds(b * SEQ + s0 + t * T, T)],
            osem.at[slot],
        )

    pe_gather(jnp.int32(0)).start()
    x_copy(jnp.int32(0)).start()
    x_copy(jnp.int32(1)).start()
    x_copy(jnp.int32(2)).start()

    def item(k, _):
        t = lax.div(k, BATCH)
        b = lax.rem(k, BATCH)
        slot = lax.rem(k, 2)
        xslot = lax.rem(k, 3)
        pslot = lax.rem(t, 2)

        @pl.when(b == 0)
        def _():
            pe_wait(t).wait()

        @pl.when(jnp.logical_and(b == 3, t + 1 < NT))
        def _():
            pe_gather(t + 1).start()

        @pl.when(k >= 2)
        def _():
            o_copy(k - 2).wait()

        x_copy(k).wait()

        @plsc.parallel_loop(0, T * (D // LANES), step=1, unroll=8)
        def add_group(g):
            r = lax.div(g, D // LANES)
            sl = pl.ds(lax.rem(g, D // LANES) * LANES, LANES)
            obuf[slot, r, sl] = xbuf[xslot, r, sl] + pebuf[pslot, r, sl]
        o_copy(k).start()

        @pl.when(k + 3 < NI)
        def _():
            x_copy(k + 3).start()

        return 0

    lax.fori_loop(0, NI, item, 0)
    o_copy(jnp.int32(NI - 2)).wait()
    o_copy(jnp.int32(NI - 1)).wait()


@jax.jit
def _pe_add(x, pos_embedding, sp16):
    batch, seq, d = x.shape
    xf = x.reshape(batch * seq, d)
    out = _sc_pe_add(xf, pos_embedding, sp16)
    return out.reshape(x.shape)


def kernel(x, pos_embedding, start_pos):
    sp16 = jnp.full((LANES,), start_pos, dtype=jnp.int32)
    return _pe_add(x, pos_embedding, sp16)


# speedup vs baseline: 1.7056x; 1.0231x over previous
import functools

import jax
import jax.numpy as jnp
from jax import lax
from jax.experimental import pallas as pl
from jax.experimental.pallas import tpu as pltpu
from jax.experimental.pallas import tpu_sc as plsc

D = 1024
SEQ = 4096
BATCH = 4
NW = 32
SW = SEQ // NW
T = 16
NT = SW // T
LANES = 16
NI = NT * BATCH

_mesh = plsc.VectorSubcoreMesh(core_axis_name="c", subcore_axis_name="s")


@functools.partial(
    pl.kernel,
    mesh=_mesh,
    out_type=jax.ShapeDtypeStruct((BATCH * SEQ, D), jnp.float32),
    scratch_types=[
        pltpu.VMEM((2, T, D), jnp.float32),
        pltpu.VMEM((3, T, D), jnp.float32),
        pltpu.VMEM((2, T, D), jnp.float32),
        pltpu.VMEM((2, T), jnp.int32),
        pltpu.VMEM((LANES,), jnp.int32),
        pltpu.SemaphoreType.DMA((2,)),
        pltpu.SemaphoreType.DMA((3,)),
        pltpu.SemaphoreType.DMA((2,)),
    ],
)
def _sc_pe_add(x_hbm, pe_hbm, sp_hbm, out_hbm, pebuf, xbuf, obuf, idxbuf,
               spbuf, gsem, xsem, osem):
    wid = lax.axis_index("s") * 2 + lax.axis_index("c")
    s0 = wid * SW
    pltpu.sync_copy(sp_hbm, spbuf)
    vsp = spbuf[...]

    def pe_gather(t):
        slot = lax.rem(t, 2)
        idxbuf[slot, :] = vsp + lax.iota(jnp.int32, LANES) + (s0 + t * T)
        return pltpu.make_async_copy(
            pe_hbm.at[idxbuf.at[slot]], pebuf.at[slot], gsem.at[slot]
        )

    def pe_wait(t):
        slot = lax.rem(t, 2)
        return pltpu.make_async_copy(
            pe_hbm.at[idxbuf.at[slot]], pebuf.at[slot], gsem.at[slot]
        )

    def x_copy(k):
        t = lax.div(k, BATCH)
        b = lax.rem(k, BATCH)
        slot = lax.rem(k, 3)
        return pltpu.make_async_copy(
            x_hbm.at[pl.ds(b * SEQ + s0 + t * T, T)],
            xbuf.at[slot],
            xsem.at[slot],
        )

    def o_copy(k):
        t = lax.div(k, BATCH)
        b = lax.rem(k, BATCH)
        slot = lax.rem(k, 2)
        return pltpu.make_async_copy(
            obuf.at[slot],
            out_hbm.at[pl.ds(b * SEQ + s0 + t * T, T)],
            osem.at[slot],
        )

    pe_gather(jnp.int32(0)).start()
    x_copy(jnp.int32(0)).start()
    x_copy(jnp.int32(1)).start()
    x_copy(jnp.int32(2)).start()

    def item(k, _):
        t = lax.div(k, BATCH)
        b = lax.rem(k, BATCH)
        slot = lax.rem(k, 2)
        xslot = lax.rem(k, 3)
        pslot = lax.rem(t, 2)

        @pl.when(b == 0)
        def _():
            pe_wait(t).wait()

        @pl.when(jnp.logical_and(b == 0, t + 1 < NT))
        def _():
            pe_gather(t + 1).start()

        @pl.when(k >= 2)
        def _():
            o_copy(k - 2).wait()

        x_copy(k).wait()

        @plsc.parallel_loop(0, T * (D // LANES), step=1, unroll=8)
        def add_group(g):
            r = lax.div(g, D // LANES)
            sl = pl.ds(lax.rem(g, D // LANES) * LANES, LANES)
            obuf[slot, r, sl] = xbuf[xslot, r, sl] + pebuf[pslot, r, sl]
        o_copy(k).start()

        @pl.when(k + 3 < NI)
        def _():
            x_copy(k + 3).start()

        return 0

    lax.fori_loop(0, NI, item, 0)
    o_copy(jnp.int32(NI - 2)).wait()
    o_copy(jnp.int32(NI - 1)).wait()


@jax.jit
def _pe_add(x, pos_embedding, sp16):
    batch, seq, d = x.shape
    xf = x.reshape(batch * seq, d)
    out = _sc_pe_add(xf, pos_embedding, sp16)
    return out.reshape(x.shape)


def kernel(x, pos_embedding, start_pos):
    sp16 = jnp.full((LANES,), start_pos, dtype=jnp.int32)
    return _pe_add(x, pos_embedding, sp16)
